# baseline (speedup 1.0000x reference)
import functools

import jax
import jax.numpy as jnp
from jax import lax
from jax.experimental import pallas as pl
from jax.experimental.pallas import tpu as pltpu
from jax.experimental.pallas import tpu_sc as plsc

N_NODES = 10000
NFEAT = 128
NHID = 128

NC = 2
NS = 16
NW = NC * NS
L = 16

N_PAD = 10240
ROWS_PER_TILE = N_PAD // NS

N_EDGES = 320000
CHUNK = 128
NCHUNK = 80
EPT = NCHUNK * CHUNK
E_PAD = EPT * NW

_mesh = plsc.VectorSubcoreMesh(core_axis_name="c", subcore_axis_name="s")


@functools.partial(
    pl.kernel,
    out_type=jax.ShapeDtypeStruct((NW, N_PAD), jnp.float32),
    mesh=_mesh,
    scratch_types=[
        pltpu.VMEM((EPT,), jnp.int32),
        pltpu.VMEM((N_PAD,), jnp.float32),
    ],
    compiler_params=pltpu.CompilerParams(needs_layout_passes=False),
)
def _deg_kernel(dst_hbm, out_hbm, idx_v, deg_v):
    c = lax.axis_index("c")
    s = lax.axis_index("s")
    wid = c * NS + s
    zeros = jnp.zeros((L,), jnp.float32)

    def zero_body(i, _):
        deg_v[pl.ds(i * L, L)] = zeros
        return 0

    lax.fori_loop(0, N_PAD // L, zero_body, 0)

    pltpu.sync_copy(dst_hbm.at[pl.ds(wid * EPT, EPT)], idx_v)
    ones = jnp.ones((L,), jnp.float32)

    def body(i, _):
        idx = idx_v[pl.ds(i * L, L)]
        plsc.addupdate_scatter(deg_v, [idx], ones)
        return 0

    lax.fori_loop(0, EPT // L, body, 0)
    pltpu.sync_copy(deg_v, out_hbm.at[wid])


def _tc1_body(xp_ref, w_ref, degp_ref, g_ref, dinv_ref):
    deg = jnp.sum(degp_ref[...], axis=0) + 1.0
    dinv = lax.rsqrt(deg)
    h = jnp.dot(xp_ref[...], w_ref[...], preferred_element_type=jnp.float32)
    g_ref[...] = h * dinv[:, None]
    dinv_ref[...] = dinv


def _tc1(xp, w, degp):
    return pl.pallas_call(
        _tc1_body,
        out_shape=(
            jax.ShapeDtypeStruct((N_PAD, NHID), jnp.float32),
            jax.ShapeDtypeStruct((N_PAD,), jnp.float32),
        ),
    )(xp, w, degp)


@functools.partial(
    pl.kernel,
    out_type=jax.ShapeDtypeStruct((NC, N_PAD, NHID), jnp.float32),
    mesh=_mesh,
    scratch_types=[
        pltpu.VMEM((NCHUNK // 2, CHUNK), jnp.int32),
        pltpu.VMEM((NCHUNK // 2, CHUNK), jnp.int32),
        pltpu.VMEM((2, CHUNK, NHID), jnp.float32),
        pltpu.VMEM_SHARED((N_PAD, NHID), jnp.float32),
        pltpu.SemaphoreType.DMA,
    ],
)
def _edge_kernel(src_hbm, dst_hbm, g_hbm, out_hbm,
                 src_v, dst_v, rows_v, acc_sh, sem):
    c = lax.axis_index("c")
    s = lax.axis_index("s")
    wid = (1 - c) * NS + s
    H = NCHUNK // 2

    zz = jnp.zeros((L,), jnp.float32)

    def zb(i, _):
        rows_v[0, i // (NHID // L), pl.ds((i % (NHID // L)) * L, L)] = zz
        return 0

    lax.fori_loop(0, CHUNK * (NHID // L), zb, 0)
    base_row = s * ROWS_PER_TILE
    for k in range(ROWS_PER_TILE // CHUNK):
        pltpu.sync_copy(rows_v.at[0],
                        acc_sh.at[pl.ds(base_row + k * CHUNK, CHUNK)])

    plsc.subcore_barrier()

    for h in range(2):
        pltpu.sync_copy(src_hbm.at[pl.ds(wid * NCHUNK + h * H, H)], src_v)
        pltpu.sync_copy(dst_hbm.at[pl.ds(wid * NCHUNK + h * H, H)], dst_v)

        pltpu.async_copy(g_hbm.at[src_v.at[0]], rows_v.at[0], sem)

        def chunk_body(j, _):
            nxt = j + 1
            pltpu.async_copy(g_hbm.at[src_v.at[nxt]], rows_v.at[nxt & 1], sem)
            pltpu.make_async_copy(g_hbm.at[src_v.at[j]], rows_v.at[j & 1],
                                  sem).wait()
            pltpu.sync_copy(rows_v.at[j & 1], acc_sh.at[dst_v.at[j]],
                            add=True)
            return 0

        lax.fori_loop(0, H - 1, chunk_body, 0)
        last = H - 1
        pltpu.make_async_copy(g_hbm.at[src_v.at[last]], rows_v.at[last & 1],
                              sem).wait()
        pltpu.sync_copy(rows_v.at[last & 1], acc_sh.at[dst_v.at[last]],
                        add=True)

    plsc.subcore_barrier()
    pltpu.sync_copy(acc_sh.at[pl.ds(base_row, ROWS_PER_TILE)],
                    out_hbm.at[c, pl.ds(base_row, ROWS_PER_TILE)])


def _tc2_body(accp_ref, g_ref, dinv_ref, b_ref, out_ref):
    ssum = accp_ref[0] + accp_ref[1] + g_ref[...]
    out_ref[...] = ssum * dinv_ref[...][:, None] + b_ref[...][None, :]


def _tc2(accp, g, dinv, b):
    return pl.pallas_call(
        _tc2_body,
        out_shape=jax.ShapeDtypeStruct((N_PAD, NHID), jnp.float32),
    )(accp, g, dinv, b)


@jax.jit
def kernel(x, edge_index, W, b):
    src = edge_index[0].astype(jnp.int32)
    dst = edge_index[1].astype(jnp.int32)
    pad = E_PAD - src.shape[0]
    src_p = jnp.concatenate([src, jnp.zeros((pad,), jnp.int32)])
    pad_dst = N_NODES + (jnp.arange(pad, dtype=jnp.int32) % (N_PAD - N_NODES))
    dst_p = jnp.concatenate([dst, pad_dst])
    src2 = src_p.reshape(NW * NCHUNK, CHUNK)
    dst2 = dst_p.reshape(NW * NCHUNK, CHUNK)

    xp = jnp.pad(x, ((0, N_PAD - x.shape[0]), (0, 0)))

    degp = _deg_kernel(dst_p)
    g, dinv = _tc1(xp, W, degp)
    accp = _edge_kernel(src2, dst2, g)
    out = _tc2(accp, g, dinv, b)
    return out[:N_NODES]


# speedup vs baseline: 3.6076x; 3.6076x over previous
import functools

import jax
import jax.numpy as jnp
from jax import lax
from jax.experimental import pallas as pl
from jax.experimental.pallas import tpu as pltpu
from jax.experimental.pallas import tpu_sc as plsc

N_NODES = 10000
NFEAT = 128
NHID = 128

NC = 2
NS = 16
NW = NC * NS
L = 16

N_PAD = 10240
ROWS_PER_TILE = N_PAD // NS

N_EDGES = 320000
CHUNK = 128
NCHUNK = 80
EPT = NCHUNK * CHUNK
E_PAD = EPT * NW

_mesh = plsc.VectorSubcoreMesh(core_axis_name="c", subcore_axis_name="s")


@functools.partial(
    pl.kernel,
    out_type=jax.ShapeDtypeStruct((NW, N_PAD), jnp.float32),
    mesh=_mesh,
    scratch_types=[
        pltpu.VMEM((EPT,), jnp.int32),
        pltpu.VMEM((N_PAD,), jnp.float32),
    ],
    compiler_params=pltpu.CompilerParams(needs_layout_passes=False),
)
def _deg_kernel(dst_hbm, out_hbm, idx_v, deg_v):
    c = lax.axis_index("c")
    s = lax.axis_index("s")
    wid = c * NS + s
    zeros = jnp.zeros((L,), jnp.float32)

    def zero_body(i, _):
        deg_v[pl.ds(i * L, L)] = zeros
        return 0

    lax.fori_loop(0, N_PAD // L, zero_body, 0)

    pltpu.sync_copy(dst_hbm.at[pl.ds(wid * EPT, EPT)], idx_v)
    ones = jnp.ones((L,), jnp.float32)

    def body(i, _):
        idx = idx_v[pl.ds(i * L, L)]
        plsc.addupdate_scatter(deg_v, [idx], ones)
        return 0

    lax.fori_loop(0, EPT // L, body, 0)
    pltpu.sync_copy(deg_v, out_hbm.at[wid])


def _tc1_body(xp_ref, w_ref, degp_ref, g_ref, dinv_ref):
    deg = jnp.sum(degp_ref[...], axis=0) + 1.0
    dinv = lax.rsqrt(deg)
    h = jnp.dot(xp_ref[...], w_ref[...], preferred_element_type=jnp.float32)
    g_ref[...] = h * dinv[:, None]
    dinv_ref[...] = dinv


def _tc1(xp, w, degp):
    return pl.pallas_call(
        _tc1_body,
        out_shape=(
            jax.ShapeDtypeStruct((N_PAD, NHID), jnp.float32),
            jax.ShapeDtypeStruct((N_PAD,), jnp.float32),
        ),
    )(xp, w, degp)


@functools.partial(
    pl.kernel,
    out_type=jax.ShapeDtypeStruct((NC, N_PAD, NHID), jnp.float32),
    mesh=_mesh,
    scratch_types=[
        pltpu.VMEM((NCHUNK // 2, CHUNK), jnp.int32),
        pltpu.VMEM((NCHUNK // 2, CHUNK), jnp.int32),
        pltpu.VMEM((2, CHUNK, NHID), jnp.float32),
        pltpu.VMEM_SHARED((N_PAD, NHID), jnp.float32),
        pltpu.SemaphoreType.DMA,
    ],
)
def _edge_kernel(src_hbm, dst_hbm, g_hbm, out_hbm,
                 src_v, dst_v, rows_v, acc_sh, sem):
    c = lax.axis_index("c")
    s = lax.axis_index("s")
    wid = c * NS + s
    H = NCHUNK // 2
    n_real = jnp.clip((N_EDGES - wid * EPT + CHUNK - 1) // CHUNK, 0, NCHUNK)

    zz = jnp.zeros((L,), jnp.float32)

    def zb(i, _):
        rows_v[0, i // (NHID // L), pl.ds((i % (NHID // L)) * L, L)] = zz
        return 0

    lax.fori_loop(0, CHUNK * (NHID // L), zb, 0)
    base_row = s * ROWS_PER_TILE
    for k in range(ROWS_PER_TILE // CHUNK):
        pltpu.sync_copy(rows_v.at[0],
                        acc_sh.at[pl.ds(base_row + k * CHUNK, CHUNK)])

    plsc.subcore_barrier()

    for h in range(2):
        nh = jnp.clip(n_real - h * H, 0, H)

        @pl.when(nh > 0)
        def _half():
            pltpu.sync_copy(src_hbm.at[pl.ds(wid * NCHUNK + h * H, H)],
                            src_v)
            pltpu.sync_copy(dst_hbm.at[pl.ds(wid * NCHUNK + h * H, H)],
                            dst_v)

            pltpu.async_copy(g_hbm.at[src_v.at[0]], rows_v.at[0], sem)

            def chunk_body(j, _):
                nxt = j + 1
                pltpu.async_copy(g_hbm.at[src_v.at[nxt]], rows_v.at[nxt & 1],
                                 sem)
                pltpu.make_async_copy(g_hbm.at[src_v.at[j]],
                                      rows_v.at[j & 1], sem).wait()
                pltpu.sync_copy(rows_v.at[j & 1], acc_sh.at[dst_v.at[j]],
                                add=True)
                return 0

            lax.fori_loop(0, nh - 1, chunk_body, 0)
            last = nh - 1
            pltpu.make_async_copy(g_hbm.at[src_v.at[last]],
                                  rows_v.at[last & 1], sem).wait()
            pltpu.sync_copy(rows_v.at[last & 1], acc_sh.at[dst_v.at[last]],
                            add=True)

    plsc.subcore_barrier()
    pltpu.sync_copy(acc_sh.at[pl.ds(base_row, ROWS_PER_TILE)],
                    out_hbm.at[c, pl.ds(base_row, ROWS_PER_TILE)])


def _tc2_body(accp_ref, g_ref, dinv_ref, b_ref, out_ref):
    ssum = accp_ref[0] + accp_ref[1] + g_ref[...]
    out_ref[...] = ssum * dinv_ref[...][:, None] + b_ref[...][None, :]


def _tc2(accp, g, dinv, b):
    return pl.pallas_call(
        _tc2_body,
        out_shape=jax.ShapeDtypeStruct((N_PAD, NHID), jnp.float32),
    )(accp, g, dinv, b)


@jax.jit
def kernel(x, edge_index, W, b):
    src = edge_index[0].astype(jnp.int32)
    dst = edge_index[1].astype(jnp.int32)
    pad = E_PAD - src.shape[0]
    src_p = jnp.concatenate([src, jnp.zeros((pad,), jnp.int32)])
    pad_dst = N_NODES + (jnp.arange(pad, dtype=jnp.int32) % (N_PAD - N_NODES))
    dst_p = jnp.concatenate([dst, pad_dst])
    src2 = src_p.reshape(NW * NCHUNK, CHUNK)
    dst2 = dst_p.reshape(NW * NCHUNK, CHUNK)

    xp = jnp.pad(x, ((0, N_PAD - x.shape[0]), (0, 0)))

    degp = _deg_kernel(dst_p)
    g, dinv = _tc1(xp, W, degp)
    accp = _edge_kernel(src2, dst2, g)
    out = _tc2(accp, g, dinv, b)
    return out[:N_NODES]
